# Initial kernel scaffold; baseline (speedup 1.0000x reference)
import jax
import jax.numpy as jnp
from jax.experimental import pallas as pl


def kernel(input_ids, W):
    raise NotImplementedError("write your pallas kernel here")


# speedup vs baseline: 1.6411x; 1.6411x over previous
import functools

import jax
import jax.numpy as jnp
from jax import lax
from jax.experimental import pallas as pl
from jax.experimental.pallas import tpu as pltpu
from jax.experimental.pallas import tpu_sc as plsc

_SUB = 128
_SUBS_PER_CHUNK = 8
_CHUNK = _SUB * _SUBS_PER_CHUNK


@functools.lru_cache(maxsize=None)
def _build_lookup(n_rows: int, d: int):
    info = plsc.get_sparse_core_info()
    nc, ns = info.num_cores, info.num_subcores
    nw = nc * ns
    assert n_rows % (nw * _CHUNK) == 0
    chunks_per_w = n_rows // (nw * _CHUNK)
    subs_per_w = chunks_per_w * _SUBS_PER_CHUNK
    mesh = plsc.VectorSubcoreMesh(core_axis_name="c", subcore_axis_name="s")

    @functools.partial(
        pl.kernel,
        mesh=mesh,
        out_type=jax.ShapeDtypeStruct((n_rows, d), jnp.float32),
        scratch_types=[
            pltpu.VMEM((_SUBS_PER_CHUNK, _SUB), jnp.int32),
            pltpu.VMEM((_CHUNK, d), jnp.float32),
            pltpu.SemaphoreType.DMA,
        ],
        compiler_params=pltpu.CompilerParams(use_tc_tiling_on_sc=False),
    )
    def lookup(ids_hbm, table_hbm, out_hbm, idx_v, rows_v, sem):
        wid = lax.axis_index("s") * nc + lax.axis_index("c")

        def chunk_body(ci, carry):
            sub_base = wid * subs_per_w + ci * _SUBS_PER_CHUNK
            pltpu.sync_copy(ids_hbm.at[pl.ds(sub_base, _SUBS_PER_CHUNK)], idx_v)
            copies = [
                pltpu.async_copy(
                    table_hbm.at[idx_v.at[j]],
                    rows_v.at[pl.ds(j * _SUB, _SUB)],
                    sem,
                )
                for j in range(_SUBS_PER_CHUNK)
            ]
            for cpy in copies:
                cpy.wait()
            pltpu.sync_copy(rows_v, out_hbm.at[pl.ds(sub_base * _SUB, _CHUNK)])
            return carry

        lax.fori_loop(0, chunks_per_w, chunk_body, 0)

    return lookup


def kernel(input_ids, W):
    bsz, seq = input_ids.shape
    d = W.shape[1]
    n_rows = bsz * seq
    ids2d = input_ids.astype(jnp.int32).reshape(n_rows // _SUB, _SUB)
    out = _build_lookup(n_rows, d)(ids2d, W)
    return out.reshape(bsz, seq, d)


# speedup vs baseline: 4.8624x; 2.9628x over previous
import functools

import jax
import jax.numpy as jnp
from jax import lax
from jax.experimental import pallas as pl
from jax.experimental.pallas import tpu as pltpu
from jax.experimental.pallas import tpu_sc as plsc

_SUB = 128
_SUBS_PER_CHUNK = 4
_CHUNK = _SUB * _SUBS_PER_CHUNK


@functools.lru_cache(maxsize=None)
def _build_lookup(n_rows: int, v: int, d: int):
    info = plsc.get_sparse_core_info()
    nc, ns = info.num_cores, info.num_subcores
    nw = nc * ns
    assert n_rows % (nw * 2 * _CHUNK) == 0
    chunks_per_w = n_rows // (nw * _CHUNK)
    subs_per_w = chunks_per_w * _SUBS_PER_CHUNK
    mesh = plsc.VectorSubcoreMesh(core_axis_name="c", subcore_axis_name="s")

    @functools.partial(
        pl.kernel,
        mesh=mesh,
        out_type=jax.ShapeDtypeStruct((n_rows, d), jnp.float32),
        scratch_types=[
            pltpu.VMEM_SHARED((v, d), jnp.float32),
            pltpu.VMEM((2, _SUBS_PER_CHUNK, _SUB), jnp.int32),
            pltpu.VMEM((2, _CHUNK, d), jnp.float32),
            pltpu.SemaphoreType.DMA,
            pltpu.SemaphoreType.DMA,
        ],
        compiler_params=pltpu.CompilerParams(use_tc_tiling_on_sc=False),
    )
    def lookup(ids_hbm, table_hbm, out_hbm, table_v, idx_v, rows_v, sem0, sem1):
        wid = lax.axis_index("s") * nc + lax.axis_index("c")

        @pl.when(lax.axis_index("s") == 0)
        def _():
            pltpu.sync_copy(table_hbm, table_v)

        plsc.subcore_barrier()
        sems = (sem0, sem1)

        def fetch_ids(ci, b):
            sub_base = wid * subs_per_w + ci * _SUBS_PER_CHUNK
            pltpu.sync_copy(ids_hbm.at[pl.ds(sub_base, _SUBS_PER_CHUNK)],
                            idx_v.at[b])

        def fire_gathers(b):
            return [
                pltpu.async_copy(
                    table_v.at[idx_v.at[b].at[j]],
                    rows_v.at[b].at[pl.ds(j * _SUB, _SUB)],
                    sems[b],
                )
                for j in range(_SUBS_PER_CHUNK)
            ]

        def out_copy(ci, b):
            pltpu.sync_copy(rows_v.at[b],
                            out_hbm.at[pl.ds((wid * subs_per_w
                                              + ci * _SUBS_PER_CHUNK) * _SUB,
                                             _CHUNK)])

        fetch_ids(0, 0)
        g0 = fire_gathers(0)

        def pair_body(p, carry):
            c0 = 2 * p
            fetch_ids(c0 + 1, 1)
            g1 = fire_gathers(1)
            for cpy in g0:
                cpy.wait()
            out_copy(c0, 0)

            @pl.when(p < chunks_per_w // 2 - 1)
            def _():
                fetch_ids(c0 + 2, 0)
                fire_gathers(0)

            for cpy in g1:
                cpy.wait()
            out_copy(c0 + 1, 1)
            return carry

        lax.fori_loop(0, chunks_per_w // 2, pair_body, 0)

    return lookup


def kernel(input_ids, W):
    bsz, seq = input_ids.shape
    v, d = W.shape
    n_rows = bsz * seq
    ids2d = input_ids.astype(jnp.int32).reshape(n_rows // _SUB, _SUB)
    out = _build_lookup(n_rows, v, d)(ids2d, W)
    return out.reshape(bsz, seq, d)
